# baseline (speedup 1.0000x reference)
import functools

import jax
import jax.numpy as jnp
from jax import lax
from jax.experimental import pallas as pl
from jax.experimental.pallas import tpu as pltpu
from jax.experimental.pallas import tpu_sc as plsc

_NBUF = 2


@functools.cache
def _make_gather(B: int, H: int, V: int, D: int):
    info = plsc.get_sparse_core_info()
    nc, ns, L = info.num_cores, info.num_subcores, info.num_lanes
    nw = nc * ns
    BT = B // 128
    bt_w = BT // nw
    HP = (H + 7) // 8 * 8
    n_blocks = H * bt_w
    nbuf = _NBUF
    n_steps = n_blocks // nbuf
    assert n_blocks % nbuf == 0
    mesh = plsc.VectorSubcoreMesh(core_axis_name="c", subcore_axis_name="s")

    @functools.partial(
        pl.kernel,
        mesh=mesh,
        out_type=jax.ShapeDtypeStruct((H, D // 8, 128, 8, 128), jnp.float32),
        scratch_types=[
            pltpu.VMEM((HP // 8, bt_w, 8, 128), jnp.int32),
            [pltpu.VMEM((128, D), jnp.float32) for _ in range(nbuf)],
            [pltpu.VMEM((D // 8, 8, 128), jnp.float32) for _ in range(nbuf)],
            [pltpu.SemaphoreType.DMA for _ in range(nbuf)],
            [pltpu.SemaphoreType.DMA for _ in range(nbuf)],
        ],
        compiler_params=pltpu.CompilerParams(
            use_tc_tiling_on_sc=False, needs_layout_passes=False
        ),
    )
    def gather_kernel(table_hbm, idx_hbm, out_hbm, slab, rows, tbuf, sem_g,
                      sem_o):
        wid = lax.axis_index("s") * nc + lax.axis_index("c")
        bt0 = wid * bt_w
        pltpu.sync_copy(idx_hbm.at[:, pl.ds(bt0, bt_w)], slab)

        iota = lax.iota(jnp.int32, L)
        row_idx = [jnp.int32(g * L) + iota for g in range(128 // L)]

        def block_jbt(q):
            j = q // bt_w
            return j, q - j * bt_w

        def gather(q, b):
            j, btl = block_jbt(q)
            pltpu.async_copy(
                table_hbm.at[slab.at[j // 8, btl, j % 8]], rows[b], sem_g[b]
            )

        def wait_gather(b):
            pltpu.make_async_copy(
                table_hbm.at[slab.at[0, 0, 0]], rows[b], sem_g[b]
            ).wait()

        def transpose(b):
            def kbody(k, _):
                col = lax.broadcast(k, (L,))
                kh = k // 8
                kl = k - kh * 8
                for g in range(128 // L):
                    v = plsc.load_gather(rows[b], [row_idx[g], col])
                    tbuf[b][kh, kl, pl.ds(g * L, L)] = v
                return _

            lax.fori_loop(0, D, kbody, 0, unroll=2)

        def writeout(q, b):
            j, btl = block_jbt(q)
            pltpu.async_copy(
                tbuf[b], out_hbm.at[j, :, bt0 + btl], sem_o[b]
            )

        def wait_out(b):
            pltpu.make_async_copy(
                tbuf[b], out_hbm.at[0, :, 0], sem_o[b]
            ).wait()

        for b in range(nbuf):
            gather(b, b)
        for b in range(nbuf):
            wait_gather(b)
            transpose(b)
            gather(b + nbuf, b)
            writeout(b, b)

        def body(s, carry):
            for b in range(nbuf):
                q = s * nbuf + b
                wait_gather(b)
                wait_out(b)
                transpose(b)

                @pl.when(q + nbuf < n_blocks)
                def _():
                    gather(q + nbuf, b)

                writeout(q, b)
            return carry

        lax.fori_loop(1, n_steps, body, 0)
        for b in range(nbuf):
            wait_out(b)

    return gather_kernel


def kernel(inputs, table):
    b, h = inputs.shape
    v, d = table.shape
    idx4 = (
        jnp.pad(inputs.astype(jnp.int32), ((0, 0), (0, -h % 8)))
        .reshape(128, b // 128, (h + 7) // 8, 8)
        .transpose(2, 0, 3, 1)
    )
    out5 = _make_gather(b, h, v, d)(table, idx4)
    return out5.transpose(2, 4, 0, 1, 3).reshape(b, h, d)


# speedup vs baseline: 1.1407x; 1.1407x over previous
import functools

import jax
import jax.numpy as jnp
from jax import lax
from jax.experimental import pallas as pl
from jax.experimental.pallas import tpu as pltpu
from jax.experimental.pallas import tpu_sc as plsc

_NBUF = 2


@functools.cache
def _make_gather(B: int, H: int, V: int, D: int):
    info = plsc.get_sparse_core_info()
    nc, ns, L = info.num_cores, info.num_subcores, info.num_lanes
    nw = nc * ns
    BT = B // 128
    bt_w = BT // nw
    HP = (H + 7) // 8 * 8
    n_blocks = H * bt_w
    nbuf = _NBUF
    n_steps = n_blocks // nbuf
    assert n_blocks % nbuf == 0
    mesh = plsc.VectorSubcoreMesh(core_axis_name="c", subcore_axis_name="s")

    @functools.partial(
        pl.kernel,
        mesh=mesh,
        out_type=jax.ShapeDtypeStruct((H, D // 8, 128, 8, 128), jnp.float32),
        scratch_types=[
            pltpu.VMEM((HP // 8, bt_w, 8, 128), jnp.int32),
            [pltpu.VMEM((128, D), jnp.float32) for _ in range(nbuf)],
            [pltpu.VMEM((D // 8, 8, 128), jnp.float32) for _ in range(nbuf)],
            [pltpu.SemaphoreType.DMA for _ in range(nbuf)],
            [pltpu.SemaphoreType.DMA for _ in range(nbuf)],
        ],
        compiler_params=pltpu.CompilerParams(
            use_tc_tiling_on_sc=False, needs_layout_passes=False
        ),
    )
    def gather_kernel(table_hbm, idx_hbm, out_hbm, slab, rows, tbuf, sem_g,
                      sem_o):
        wid = lax.axis_index("s") * nc + lax.axis_index("c")
        bt0 = wid * bt_w
        pltpu.sync_copy(idx_hbm.at[:, pl.ds(bt0, bt_w)], slab)

        iota = lax.iota(jnp.int32, L)
        row_idx = [jnp.int32(g * L) + iota for g in range(128 // L)]

        def block_jbt(q):
            j = q // bt_w
            return j, q - j * bt_w

        def gather(q, b):
            j, btl = block_jbt(q)
            pltpu.async_copy(
                table_hbm.at[slab.at[j // 8, btl, j % 8]], rows[b], sem_g[b]
            )

        def wait_gather(b):
            pltpu.make_async_copy(
                table_hbm.at[slab.at[0, 0, 0]], rows[b], sem_g[b]
            ).wait()

        def transpose(b):
            def kbody(k, _):
                col = lax.broadcast(k, (L,))
                kh = k // 8
                kl = k - kh * 8
                vs = [
                    plsc.load_gather(rows[b], [row_idx[g], col])
                    for g in range(128 // L)
                ]
                for g, v in enumerate(vs):
                    tbuf[b][kh, kl, pl.ds(g * L, L)] = v
                return _

            lax.fori_loop(0, D, kbody, 0, unroll=4)

        def writeout(q, b):
            j, btl = block_jbt(q)
            pltpu.async_copy(
                tbuf[b], out_hbm.at[j, :, bt0 + btl], sem_o[b]
            )

        def wait_out(b):
            pltpu.make_async_copy(
                tbuf[b], out_hbm.at[0, :, 0], sem_o[b]
            ).wait()

        for b in range(nbuf):
            gather(b, b)
        for b in range(nbuf):
            wait_gather(b)
            transpose(b)
            gather(b + nbuf, b)
            writeout(b, b)

        def body(s, carry):
            for b in range(nbuf):
                q = s * nbuf + b
                wait_gather(b)
                wait_out(b)
                transpose(b)

                @pl.when(q + nbuf < n_blocks)
                def _():
                    gather(q + nbuf, b)

                writeout(q, b)
            return carry

        lax.fori_loop(1, n_steps, body, 0)
        for b in range(nbuf):
            wait_out(b)

    return gather_kernel


def kernel(inputs, table):
    b, h = inputs.shape
    v, d = table.shape
    idx4 = (
        jnp.pad(inputs.astype(jnp.int32), ((0, 0), (0, -h % 8)))
        .reshape(128, b // 128, (h + 7) // 8, 8)
        .transpose(2, 0, 3, 1)
    )
    out5 = _make_gather(b, h, v, d)(table, idx4)
    return out5.transpose(2, 4, 0, 1, 3).reshape(b, h, d)


# speedup vs baseline: 1.9529x; 1.7120x over previous
import functools

import jax
import jax.numpy as jnp
from jax import lax
from jax.experimental import pallas as pl
from jax.experimental.pallas import tpu as pltpu
from jax.experimental.pallas import tpu_sc as plsc

_NBUF = 4


@functools.cache
def _make_gather(B: int, H: int, V: int, D: int):
    info = plsc.get_sparse_core_info()
    nc, ns, L = info.num_cores, info.num_subcores, info.num_lanes
    nw = nc * ns
    BT = B // 128
    bt_w = BT // nw
    HP = (H + 7) // 8 * 8
    n_blocks = H * bt_w
    nbuf = _NBUF
    n_steps = n_blocks // nbuf
    assert n_blocks % nbuf == 0
    mesh = plsc.VectorSubcoreMesh(core_axis_name="c", subcore_axis_name="s")

    @functools.partial(
        pl.kernel,
        mesh=mesh,
        out_type=jax.ShapeDtypeStruct((B, HP // 8, 8, 128), jnp.float32),
        scratch_types=[
            pltpu.VMEM((HP // 8, bt_w, 8, 128), jnp.int32),
            [pltpu.VMEM((128, D), jnp.float32) for _ in range(nbuf)],
            [pltpu.SemaphoreType.DMA for _ in range(nbuf)],
            [pltpu.SemaphoreType.DMA for _ in range(nbuf)],
        ],
        compiler_params=pltpu.CompilerParams(
            use_tc_tiling_on_sc=False, needs_layout_passes=False
        ),
    )
    def gather_kernel(table_hbm, idx_hbm, out_hbm, slab, rows, sem_g, sem_o):
        wid = lax.axis_index("s") * nc + lax.axis_index("c")
        bt0 = wid * bt_w
        pltpu.sync_copy(idx_hbm.at[:, pl.ds(bt0, bt_w)], slab)

        def block_jbt(q):
            j = q // bt_w
            return j, q - j * bt_w

        def gather(q, b):
            j, btl = block_jbt(q)
            pltpu.async_copy(
                table_hbm.at[slab.at[j // 8, btl, j % 8]], rows[b], sem_g[b]
            )

        def wait_gather(b):
            pltpu.make_async_copy(
                table_hbm.at[slab.at[0, 0, 0]], rows[b], sem_g[b]
            ).wait()

        def writeout(q, b):
            j, btl = block_jbt(q)
            pltpu.async_copy(
                rows[b],
                out_hbm.at[pl.ds((bt0 + btl) * 128, 128), j // 8, j % 8,
                           pl.ds(0, D)],
                sem_o[b],
            )

        def wait_out(b):
            pltpu.make_async_copy(
                rows[b],
                out_hbm.at[pl.ds(0, 128), 0, 0, pl.ds(0, D)],
                sem_o[b],
            ).wait()

        gather(0, 0)

        def body(s, carry):
            for b in range(nbuf):
                q = s * nbuf + b
                bn = (b + 1) % nbuf

                @pl.when(q + 1 < n_blocks)
                def _():
                    @pl.when(q >= nbuf - 1)
                    def _():
                        wait_out(bn)

                    gather(q + 1, bn)

                wait_gather(b)
                writeout(q, b)
            return carry

        lax.fori_loop(0, n_steps, body, 0)
        for b in range(nbuf):
            wait_out(b)

    return gather_kernel


def kernel(inputs, table):
    b, h = inputs.shape
    v, d = table.shape
    hp = (h + 7) // 8 * 8
    idx4 = (
        jnp.pad(inputs.astype(jnp.int32), ((0, 0), (0, -h % 8)))
        .reshape(b // 128, 128, hp // 8, 8)
        .transpose(2, 0, 3, 1)
    )
    out4 = _make_gather(b, h, v, d)(table, idx4)
    return out4.reshape(b, hp, 128)[:, :h, :d]


# speedup vs baseline: 1.9564x; 1.0018x over previous
import functools

import jax
import jax.numpy as jnp
from jax import lax
from jax.experimental import pallas as pl
from jax.experimental.pallas import tpu as pltpu
from jax.experimental.pallas import tpu_sc as plsc

_NBUF = 8


@functools.cache
def _make_gather(B: int, H: int, V: int, D: int):
    info = plsc.get_sparse_core_info()
    nc, ns, L = info.num_cores, info.num_subcores, info.num_lanes
    nw = nc * ns
    BT = B // 128
    bt_w = BT // nw
    HP = (H + 7) // 8 * 8
    n_blocks = H * bt_w
    nbuf = _NBUF
    n_steps = n_blocks // nbuf
    assert n_blocks % nbuf == 0
    mesh = plsc.VectorSubcoreMesh(core_axis_name="c", subcore_axis_name="s")

    @functools.partial(
        pl.kernel,
        mesh=mesh,
        out_type=jax.ShapeDtypeStruct((B, HP // 8, 8, 128), jnp.float32),
        scratch_types=[
            pltpu.VMEM((HP // 8, bt_w, 8, 128), jnp.int32),
            [pltpu.VMEM((128, D), jnp.float32) for _ in range(nbuf)],
            [pltpu.SemaphoreType.DMA for _ in range(nbuf)],
            [pltpu.SemaphoreType.DMA for _ in range(nbuf)],
        ],
        compiler_params=pltpu.CompilerParams(
            use_tc_tiling_on_sc=False, needs_layout_passes=False
        ),
    )
    def gather_kernel(table_hbm, idx_hbm, out_hbm, slab, rows, sem_g, sem_o):
        wid = lax.axis_index("s") * nc + lax.axis_index("c")
        bt0 = wid * bt_w
        pltpu.sync_copy(idx_hbm.at[:, pl.ds(bt0, bt_w)], slab)

        def block_jbt(q):
            j = q // bt_w
            return j, q - j * bt_w

        def gather(q, b):
            j, btl = block_jbt(q)
            pltpu.async_copy(
                table_hbm.at[slab.at[j // 8, btl, j % 8]], rows[b], sem_g[b]
            )

        def wait_gather(b):
            pltpu.make_async_copy(
                table_hbm.at[slab.at[0, 0, 0]], rows[b], sem_g[b]
            ).wait()

        def writeout(q, b):
            j, btl = block_jbt(q)
            pltpu.async_copy(
                rows[b],
                out_hbm.at[pl.ds((bt0 + btl) * 128, 128), j // 8, j % 8,
                           pl.ds(0, D)],
                sem_o[b],
            )

        def wait_out(b):
            pltpu.make_async_copy(
                rows[b],
                out_hbm.at[pl.ds(0, 128), 0, 0, pl.ds(0, D)],
                sem_o[b],
            ).wait()

        gather(0, 0)

        def body(s, carry):
            for b in range(nbuf):
                q = s * nbuf + b
                bn = (b + 1) % nbuf

                @pl.when(q + 1 < n_blocks)
                def _():
                    @pl.when(q >= nbuf - 1)
                    def _():
                        wait_out(bn)

                    gather(q + 1, bn)

                wait_gather(b)
                writeout(q, b)
            return carry

        lax.fori_loop(0, n_steps, body, 0)
        for b in range(nbuf):
            wait_out(b)

    return gather_kernel


def kernel(inputs, table):
    b, h = inputs.shape
    v, d = table.shape
    hp = (h + 7) // 8 * 8
    idx4 = (
        jnp.pad(inputs.astype(jnp.int32), ((0, 0), (0, -h % 8)))
        .reshape(b // 128, 128, hp // 8, 8)
        .transpose(2, 0, 3, 1)
    )
    out4 = _make_gather(b, h, v, d)(table, idx4)
    return out4.reshape(b, hp, 128)[:, :h, :d]
